# baseline (speedup 1.0000x reference)
import functools

import jax
import jax.numpy as jnp
from jax import lax
from jax.experimental import pallas as pl
from jax.experimental.pallas import tpu as pltpu
from jax.experimental.pallas import tpu_sc as plsc

B = 16384
E = 32
NC, NS = 2, 16
NW = NC * NS
BPW = B // NW
IW = 128
KI = BPW // IW

RB = 2048
NBLK = B // RB


CHUNK = 32
NCH = BPW // CHUNK
L = 16


def _sc_gather4(uid_h, iid_h, gu_t3, gi_t3, mu_t3, mi_t3):
  mesh = plsc.VectorSubcoreMesh(core_axis_name="c", subcore_axis_name="s")

  @functools.partial(
      pl.kernel,
      mesh=mesh,
      out_type=[jax.ShapeDtypeStruct((B, E), jnp.float32)] * 4,
      scratch_types=[
          pltpu.VMEM((BPW,), jnp.int32),
          pltpu.VMEM((BPW,), jnp.int32),
          pltpu.VMEM((L, 8, E), jnp.float32),
          pltpu.VMEM((L, 8, E), jnp.float32),
          pltpu.VMEM((BPW, E), jnp.float32),
          pltpu.SemaphoreType.DMA,
          pltpu.SemaphoreType.DMA,
      ],
  )
  def k(uid_hb, iid_hb, gut_h, git_h, mut_h, mit_h,
        gu_o, gi_o, mu_o, mi_o,
        uid_v, iid_v, buf_a, buf_b, rows_v, sem_a, sem_b):
    wid = lax.axis_index("s") * NC + lax.axis_index("c")
    base = wid * BPW
    pltpu.sync_copy(uid_hb.at[pl.ds(base, BPW)], uid_v)
    pltpu.sync_copy(iid_hb.at[pl.ds(base, BPW)], iid_v)
    out_sl = pl.ds(base, BPW)
    for tab_h, ids_v, out_h in ((gut_h, uid_v, gu_o), (git_h, iid_v, gi_o),
                                (mut_h, uid_v, mu_o), (mit_h, iid_v, mi_o)):
      def chunk_body(c, _, tab_h=tab_h, ids_v=ids_v):
        ida = ids_v[pl.ds(c * 2 * L, L)]
        idb = ids_v[pl.ds(c * 2 * L + L, L)]
        cps_a, cps_b = [], []
        for l in range(L):
          t = lax.shift_right_logical(ida[l], 3)
          cps_a.append(pltpu.async_copy(
              tab_h.at[pl.ds(t * 8, 8)], buf_a.at[l], sem_a))
        for l in range(L):
          t = lax.shift_right_logical(idb[l], 3)
          cps_b.append(pltpu.async_copy(
              tab_h.at[pl.ds(t * 8, 8)], buf_b.at[l], sem_b))
        for buf, ids16, cps, off in ((buf_a, ida, cps_a, 0),
                                     (buf_b, idb, cps_b, L)):
          for cp in cps:
            cp.wait()
          for l in range(L):
            r = lax.bitwise_and(ids16[l], 7)
            i = c * 2 * L + off + l
            rows_v[i, pl.ds(0, L)] = buf[l, r, pl.ds(0, L)]
            rows_v[i, pl.ds(L, L)] = buf[l, r, pl.ds(L, L)]
        return 0

      lax.fori_loop(0, BPW // (2 * L), chunk_body, 0)
      pltpu.sync_copy(rows_v, out_h.at[out_sl])

  return k(uid_h, iid_h, gu_t3, gi_t3, mu_t3, mi_t3)


def _tc_body(gu_r, gi_r, mu_r, mi_r, w1_r, b1_r, w2_r, b2_r, wf_r, bf_r, out_r):
  w1 = w1_r[...]
  h1 = lax.dot_general(mu_r[...], w1[:, :E], (((1,), (1,)), ((), ())),
                       preferred_element_type=jnp.float32)
  h1 = h1 + lax.dot_general(mi_r[...], w1[:, E:], (((1,), (1,)), ((), ())),
                            preferred_element_type=jnp.float32)
  h1 = jnp.maximum(h1 + b1_r[...], 0.0)
  h2 = lax.dot_general(h1, w2_r[...], (((1,), (1,)), ((), ())),
                       preferred_element_type=jnp.float32)
  h2 = jnp.maximum(h2 + b2_r[...], 0.0)
  gmf = gu_r[...] * gi_r[...]
  wf = wf_r[...]
  p = lax.dot_general(gmf, wf[:, :E], (((1,), (1,)), ((), ())),
                      preferred_element_type=jnp.float32)
  p = p + lax.dot_general(h2, wf[:, E:], (((1,), (1,)), ((), ())),
                          preferred_element_type=jnp.float32)
  out_r[...] = jax.nn.sigmoid(p + bf_r[...])


def _tc_mlp(gu, gi, mu, mi, W1, b1, W2, b2, Wf, bf):
  row = pl.BlockSpec((RB, E), lambda i: (i, 0))
  full = lambda a: pl.BlockSpec(a.shape, lambda i: (0,) * a.ndim)
  out = pl.pallas_call(
      _tc_body,
      grid=(NBLK,),
      in_specs=[row, row, row, row,
                full(W1), full(b1), full(W2), full(b2), full(Wf), full(bf)],
      out_specs=pl.BlockSpec((RB, 1), lambda i: (i, 0)),
      out_shape=jax.ShapeDtypeStruct((B, 1), jnp.float32),
  )(gu, gi, mu, mi, W1, b1, W2, b2, Wf, bf)
  return out.reshape(B)


def kernel(user_ids, item_ids, gmf_user_table, gmf_item_table,
           mlp_user_table, mlp_item_table, W1, b1, W2, b2, Wf, bf):
  uid = user_ids.astype(jnp.int32)
  iid = item_ids.astype(jnp.int32)
  gu, gi, mu, mi = _sc_gather4(uid, iid, gmf_user_table, gmf_item_table,
                               mlp_user_table, mlp_item_table)
  return _tc_mlp(gu, gi, mu, mi,
                 W1, b1.reshape(1, 64), W2, b2.reshape(1, 32),
                 Wf, bf.reshape(1, 1))


# speedup vs baseline: 1.0089x; 1.0089x over previous
import functools

import jax
import jax.numpy as jnp
from jax import lax
from jax.experimental import pallas as pl
from jax.experimental.pallas import tpu as pltpu
from jax.experimental.pallas import tpu_sc as plsc

B = 16384
E = 32
NC, NS = 2, 16
NW = NC * NS
BPW = B // NW
IW = 128
KI = BPW // IW

RB = 2048
NBLK = B // RB


L = 16
W = 64


def _sc_gather4(uid_h, iid_h, gu_t3, gi_t3, mu_t3, mi_t3):
  mesh = plsc.VectorSubcoreMesh(core_axis_name="c", subcore_axis_name="s")

  @functools.partial(
      pl.kernel,
      mesh=mesh,
      out_type=[jax.ShapeDtypeStruct((B, E), jnp.float32)] * 4,
      scratch_types=[
          pltpu.VMEM((BPW,), jnp.int32),
          pltpu.VMEM((BPW,), jnp.int32),
          pltpu.VMEM((W, 8, E), jnp.float32),
          pltpu.VMEM((W, E), jnp.float32),
          pltpu.SemaphoreType.DMA,
      ],
  )
  def k(uid_hb, iid_hb, gut_h, git_h, mut_h, mit_h,
        gu_o, gi_o, mu_o, mi_o,
        uid_v, iid_v, tiles_v, rows_v, sem):
    wid = lax.axis_index("s") * NC + lax.axis_index("c")
    base = wid * BPW
    pltpu.sync_copy(uid_hb.at[pl.ds(base, BPW)], uid_v)
    pltpu.sync_copy(iid_hb.at[pl.ds(base, BPW)], iid_v)
    for tab_h, ids_v, out_h in ((gut_h, uid_v, gu_o), (git_h, iid_v, gi_o),
                                (mut_h, uid_v, mu_o), (mit_h, iid_v, mi_o)):
      def wave_body(w, _, tab_h=tab_h, ids_v=ids_v, out_h=out_h):
        idvs = [ids_v[pl.ds(w * W + g * L, L)] for g in range(W // L)]
        cps = []
        for g, idv in enumerate(idvs):
          for l in range(L):
            t = lax.shift_right_logical(idv[l], 3)
            cps.append(pltpu.async_copy(tab_h.at[pl.ds(t * 8, 8)],
                                        tiles_v.at[g * L + l], sem))
        for cp in cps:
          cp.wait()
        for g, idv in enumerate(idvs):
          for l in range(L):
            r = lax.bitwise_and(idv[l], 7)
            i = g * L + l
            rows_v[i, pl.ds(0, L)] = tiles_v[i, r, pl.ds(0, L)]
            rows_v[i, pl.ds(L, L)] = tiles_v[i, r, pl.ds(L, L)]
        pltpu.sync_copy(rows_v, out_h.at[pl.ds(base + w * W, W)])
        return 0

      lax.fori_loop(0, BPW // W, wave_body, 0)

  return k(uid_h, iid_h, gu_t3, gi_t3, mu_t3, mi_t3)


def _tc_body(gu_r, gi_r, mu_r, mi_r, w1_r, b1_r, w2_r, b2_r, wf_r, bf_r, out_r):
  w1 = w1_r[...]
  h1 = lax.dot_general(mu_r[...], w1[:, :E], (((1,), (1,)), ((), ())),
                       preferred_element_type=jnp.float32)
  h1 = h1 + lax.dot_general(mi_r[...], w1[:, E:], (((1,), (1,)), ((), ())),
                            preferred_element_type=jnp.float32)
  h1 = jnp.maximum(h1 + b1_r[...], 0.0)
  h2 = lax.dot_general(h1, w2_r[...], (((1,), (1,)), ((), ())),
                       preferred_element_type=jnp.float32)
  h2 = jnp.maximum(h2 + b2_r[...], 0.0)
  gmf = gu_r[...] * gi_r[...]
  wf = wf_r[...]
  p = lax.dot_general(gmf, wf[:, :E], (((1,), (1,)), ((), ())),
                      preferred_element_type=jnp.float32)
  p = p + lax.dot_general(h2, wf[:, E:], (((1,), (1,)), ((), ())),
                          preferred_element_type=jnp.float32)
  out_r[...] = jax.nn.sigmoid(p + bf_r[...])


def _tc_mlp(gu, gi, mu, mi, W1, b1, W2, b2, Wf, bf):
  row = pl.BlockSpec((RB, E), lambda i: (i, 0))
  full = lambda a: pl.BlockSpec(a.shape, lambda i: (0,) * a.ndim)
  out = pl.pallas_call(
      _tc_body,
      grid=(NBLK,),
      in_specs=[row, row, row, row,
                full(W1), full(b1), full(W2), full(b2), full(Wf), full(bf)],
      out_specs=pl.BlockSpec((RB, 1), lambda i: (i, 0)),
      out_shape=jax.ShapeDtypeStruct((B, 1), jnp.float32),
  )(gu, gi, mu, mi, W1, b1, W2, b2, Wf, bf)
  return out.reshape(B)


def kernel(user_ids, item_ids, gmf_user_table, gmf_item_table,
           mlp_user_table, mlp_item_table, W1, b1, W2, b2, Wf, bf):
  uid = user_ids.astype(jnp.int32)
  iid = item_ids.astype(jnp.int32)
  gu, gi, mu, mi = _sc_gather4(uid, iid, gmf_user_table, gmf_item_table,
                               mlp_user_table, mlp_item_table)
  return _tc_mlp(gu, gi, mu, mi,
                 W1, b1.reshape(1, 64), W2, b2.reshape(1, 32),
                 Wf, bf.reshape(1, 1))


# speedup vs baseline: 1.9125x; 1.8956x over previous
import functools

import jax
import jax.numpy as jnp
from jax import lax
from jax.experimental import pallas as pl
from jax.experimental.pallas import tpu as pltpu
from jax.experimental.pallas import tpu_sc as plsc

B = 16384
E = 32
NC, NS = 2, 16
NW = NC * NS
BPW = B // NW
L = 16

RB = 2048
NBLK = B // RB


def _sc_gather4(uid_h, iid_h, gu_t3, gi_t3, mu_t3, mi_t3):
  mesh = plsc.VectorSubcoreMesh(core_axis_name="c", subcore_axis_name="s")

  @functools.partial(
      pl.kernel,
      mesh=mesh,
      out_type=[jax.ShapeDtypeStruct((B, E), jnp.float32)] * 4,
      scratch_types=[
          pltpu.VMEM((BPW,), jnp.int32),
          pltpu.VMEM((BPW,), jnp.int32),
          pltpu.VMEM((BPW, E), jnp.float32),
          pltpu.SemaphoreType.DMA,
      ],
  )
  def k(uid_hb, iid_hb, gut_h, git_h, mut_h, mit_h,
        gu_o, gi_o, mu_o, mi_o,
        uid_v, iid_v, rows_v, sem):
    wid = lax.axis_index("s") * NC + lax.axis_index("c")
    base = wid * BPW
    pltpu.sync_copy(uid_hb.at[pl.ds(base, BPW)], uid_v)
    pltpu.sync_copy(iid_hb.at[pl.ds(base, BPW)], iid_v)
    out_sl = pl.ds(base, BPW)
    for tab_h, ids_v, out_h in ((gut_h, uid_v, gu_o), (git_h, iid_v, gi_o),
                                (mut_h, uid_v, mu_o), (mit_h, iid_v, mi_o)):
      def grp_body(g, _, tab_h=tab_h, ids_v=ids_v):
        idv = ids_v[pl.ds(g * L, L)]
        for l in range(L):
          idx = idv[l]
          t = lax.shift_right_logical(idx, 3)
          r = lax.bitwise_and(idx, 7)
          pltpu.async_copy(tab_h.at[t, r], rows_v.at[g * L + l], sem)
        return 0

      lax.fori_loop(0, BPW // L, grp_body, 0)
      pltpu.make_async_copy(out_h.at[out_sl], rows_v, sem).wait()
      pltpu.sync_copy(rows_v, out_h.at[out_sl])

  return k(uid_h, iid_h, gu_t3, gi_t3, mu_t3, mi_t3)


def _tc_body(gu_r, gi_r, mu_r, mi_r, w1_r, b1_r, w2_r, b2_r, wf_r, bf_r, out_r):
  w1 = w1_r[...]
  h1 = lax.dot_general(mu_r[...], w1[:, :E], (((1,), (1,)), ((), ())),
                       preferred_element_type=jnp.float32)
  h1 = h1 + lax.dot_general(mi_r[...], w1[:, E:], (((1,), (1,)), ((), ())),
                            preferred_element_type=jnp.float32)
  h1 = jnp.maximum(h1 + b1_r[...], 0.0)
  h2 = lax.dot_general(h1, w2_r[...], (((1,), (1,)), ((), ())),
                       preferred_element_type=jnp.float32)
  h2 = jnp.maximum(h2 + b2_r[...], 0.0)
  gmf = gu_r[...] * gi_r[...]
  wf = wf_r[...]
  p = lax.dot_general(gmf, wf[:, :E], (((1,), (1,)), ((), ())),
                      preferred_element_type=jnp.float32)
  p = p + lax.dot_general(h2, wf[:, E:], (((1,), (1,)), ((), ())),
                          preferred_element_type=jnp.float32)
  out_r[...] = jax.nn.sigmoid(p + bf_r[...])


def _tc_mlp(gu, gi, mu, mi, W1, b1, W2, b2, Wf, bf):
  row = pl.BlockSpec((RB, E), lambda i: (i, 0))
  full = lambda a: pl.BlockSpec(a.shape, lambda i: (0,) * a.ndim)
  out = pl.pallas_call(
      _tc_body,
      grid=(NBLK,),
      in_specs=[row, row, row, row,
                full(W1), full(b1), full(W2), full(b2), full(Wf), full(bf)],
      out_specs=pl.BlockSpec((RB, 1), lambda i: (i, 0)),
      out_shape=jax.ShapeDtypeStruct((B, 1), jnp.float32),
  )(gu, gi, mu, mi, W1, b1, W2, b2, Wf, bf)
  return out.reshape(B)


def kernel(user_ids, item_ids, gmf_user_table, gmf_item_table,
           mlp_user_table, mlp_item_table, W1, b1, W2, b2, Wf, bf):
  uid = user_ids.astype(jnp.int32)
  iid = item_ids.astype(jnp.int32)
  t3 = lambda t: t.reshape(t.shape[0] // 8, 8, E)
  gu, gi, mu, mi = _sc_gather4(uid, iid,
                               t3(gmf_user_table), t3(gmf_item_table),
                               t3(mlp_user_table), t3(mlp_item_table))
  return _tc_mlp(gu, gi, mu, mi,
                 W1, b1.reshape(1, 64), W2, b2.reshape(1, 32),
                 Wf, bf.reshape(1, 1))
